# baseline (speedup 1.0000x reference)
import functools

import jax
import jax.numpy as jnp
from jax import lax
from jax.experimental import pallas as pl
from jax.experimental.pallas import tpu as pltpu
from jax.experimental.pallas import tpu_sc as plsc

N = 10000
E = 320000
D = 128
LEAKY = 0.2

NC = 2
NS = 16
NW = NC * NS
EPW = E // NW
CHUNK = 80
HALF = CHUNK // 2
NCHUNKS = EPW // CHUNK
NPAD = 10240
RPT = NPAD // NS
UNROLL = 6
BODY_ITERS = (NCHUNKS - 1) // UNROLL
TAIL_START = 1 + BODY_ITERS * UNROLL


@functools.cache
def _build_spmm():
    mesh = plsc.VectorSubcoreMesh(core_axis_name="c", subcore_axis_name="s")

    @functools.partial(
        pl.kernel,
        out_type=jax.ShapeDtypeStruct((NC, NPAD, D), jnp.float32),
        mesh=mesh,
        scratch_types=[
            [pltpu.VMEM((CHUNK,), jnp.int32) for _ in range(3)],
            [pltpu.VMEM((CHUNK,), jnp.int32) for _ in range(3)],
            [pltpu.VMEM((CHUNK,), jnp.float32) for _ in range(3)],
            [pltpu.VMEM((CHUNK, D), jnp.float32) for _ in range(2)],
            pltpu.VMEM_SHARED((NPAD, D), jnp.float32),
            pltpu.SemaphoreType.DMA,
            pltpu.SemaphoreType.DMA,
            pltpu.SemaphoreType.DMA,
        ],
        compiler_params=pltpu.CompilerParams(needs_layout_passes=False),
    )
    def spmm(adj_hbm, val_hbm, emb_hbm, out_hbm,
             srcs, dsts, vals, rows, acc, sem_e, sem_g, sem_s):
        c = lax.axis_index("c")
        s = lax.axis_index("s")
        base = (s * NC + c) * EPW

        def zero_row(r, carry):
            for k in range(D // 16):
                rows[0][r, pl.ds(k * 16, 16)] = jnp.zeros((16,), jnp.float32)
            return carry

        lax.fori_loop(0, CHUNK, zero_row, 0)
        for j in range(RPT // CHUNK):
            pltpu.sync_copy(rows[0],
                            acc.at[pl.ds(s * RPT + j * CHUNK, CHUNK)])
        plsc.subcore_barrier()

        def issue_idx(j, t, d):
            eb = base + j * CHUNK
            pltpu.async_copy(adj_hbm.at[pl.ds(E + eb, CHUNK)], srcs[t], sem_e)
            pltpu.async_copy(adj_hbm.at[pl.ds(eb, CHUNK)], dsts[d], sem_e)
            pltpu.async_copy(val_hbm.at[pl.ds(eb, CHUNK)], vals[t], sem_e)

        def wait_idx(t, d):
            pltpu.make_async_copy(
                adj_hbm.at[pl.ds(0, CHUNK)], srcs[t], sem_e).wait()
            pltpu.make_async_copy(
                adj_hbm.at[pl.ds(0, CHUNK)], dsts[d], sem_e).wait()
            pltpu.make_async_copy(
                val_hbm.at[pl.ds(0, CHUNK)], vals[t], sem_e).wait()

        def wait_scatter(b, d):
            for h in range(2):
                pltpu.make_async_copy(
                    rows[b].at[pl.ds(h * HALF, HALF)],
                    acc.at[dsts[d].at[pl.ds(h * HALF, HALF)]], sem_s).wait()

        def step(j, b, t, d, *, first=False, issue_next=True, prefetch=True):
            if issue_next:
                wait_idx((t + 1) % 3, (d + 1) % 3)
            if not first:
                wait_scatter(1 - b, (d + 2) % 3)
            if issue_next:
                pltpu.async_copy(
                    emb_hbm.at[srcs[(t + 1) % 3]], rows[1 - b], sem_g)
            if prefetch:
                issue_idx(j + 2, (t + 2) % 3, (d + 2) % 3)
            pltpu.make_async_copy(emb_hbm.at[srcs[t]], rows[b], sem_g).wait()

            for h in range(2):
                @plsc.parallel_loop(h * HALF, (h + 1) * HALF, 1, unroll=4)
                def scale_row(r):
                    vsplat = plsc.load_gather(
                        vals[t], [jnp.full((16,), r, jnp.int32)])
                    for k in range(D // 16):
                        sl = pl.ds(k * 16, 16)
                        rows[b][r, sl] = rows[b][r, sl] * vsplat

                pltpu.async_copy(
                    rows[b].at[pl.ds(h * HALF, HALF)],
                    acc.at[dsts[d].at[pl.ds(h * HALF, HALF)]],
                    sem_s, add=True)

        issue_idx(0, 0, 0)
        wait_idx(0, 0)
        pltpu.async_copy(emb_hbm.at[srcs[0]], rows[0], sem_g)
        issue_idx(1, 1, 1)

        step(0, 0, 0, 0, first=True)

        def six(p, carry):
            for u in range(UNROLL):
                j = (1 + u) + p * UNROLL
                step(j, (1 + u) % 2, (1 + u) % 3, (1 + u) % 3)
            return carry

        lax.fori_loop(0, BODY_ITERS, six, 0)

        for j in range(TAIL_START, NCHUNKS):
            step(j, j % 2, j % 3, j % 3,
                 issue_next=(j + 1 < NCHUNKS),
                 prefetch=(j + 2 < NCHUNKS))
        wait_scatter((NCHUNKS - 1) % 2, (NCHUNKS - 1) % 3)
        plsc.subcore_barrier()

        for j in range(RPT // CHUNK):
            start = s * RPT + j * CHUNK
            pltpu.sync_copy(acc.at[pl.ds(start, CHUNK)],
                            out_hbm.at[c, pl.ds(start, CHUNK)])

    return spmm


_COMBINE_BLK = 2000


def _combine_body(p_ref, o_ref):
    x = p_ref[0] + p_ref[1]
    o_ref[...] = jnp.where(x >= 0, x, LEAKY * x)


@functools.cache
def _build_combine():
    return pl.pallas_call(
        _combine_body,
        grid=(N // _COMBINE_BLK,),
        in_specs=[pl.BlockSpec((NC, _COMBINE_BLK, D), lambda i: (0, i, 0))],
        out_specs=pl.BlockSpec((_COMBINE_BLK, D), lambda i: (i, 0)),
        out_shape=jax.ShapeDtypeStruct((N, D), jnp.float32),
    )


def kernel(adj_indices, adj_values, embeds):
    adj_flat = adj_indices.astype(jnp.int32).reshape(-1)
    partials = _build_spmm()(adj_flat, adj_values, embeds)
    return _build_combine()(partials)


# speedup vs baseline: 1.0239x; 1.0239x over previous
import functools

import jax
import jax.numpy as jnp
from jax import lax
from jax.experimental import pallas as pl
from jax.experimental.pallas import tpu as pltpu
from jax.experimental.pallas import tpu_sc as plsc

N = 10000
E = 320000
D = 128
LEAKY = 0.2

NC = 2
NS = 16
NW = NC * NS
EPW = E // NW
CHUNK = 80
HALF = CHUNK // 2
NCHUNKS = EPW // CHUNK
NPAD = 10240
RPT = NPAD // NS
UNROLL = 6
BODY_ITERS = (NCHUNKS - 1) // UNROLL
TAIL_START = 1 + BODY_ITERS * UNROLL


@functools.cache
def _build_spmm():
    mesh = plsc.VectorSubcoreMesh(core_axis_name="c", subcore_axis_name="s")

    @functools.partial(
        pl.kernel,
        out_type=jax.ShapeDtypeStruct((NC, NPAD, D), jnp.float32),
        mesh=mesh,
        scratch_types=[
            [pltpu.VMEM((CHUNK,), jnp.int32) for _ in range(3)],
            [pltpu.VMEM((CHUNK,), jnp.int32) for _ in range(3)],
            [pltpu.VMEM((CHUNK,), jnp.float32) for _ in range(3)],
            [pltpu.VMEM((CHUNK, D), jnp.float32) for _ in range(2)],
            pltpu.VMEM_SHARED((NPAD, D), jnp.float32),
            pltpu.SemaphoreType.DMA,
            pltpu.SemaphoreType.DMA,
            pltpu.SemaphoreType.DMA,
        ],
        compiler_params=pltpu.CompilerParams(needs_layout_passes=False),
    )
    def spmm(adj_hbm, val_hbm, emb_hbm, out_hbm,
             srcs, dsts, vals, rows, acc, sem_e, sem_g, sem_s):
        c = lax.axis_index("c")
        s = lax.axis_index("s")
        base = (s * NC + c) * EPW

        def zero_row(r, carry):
            for k in range(D // 16):
                rows[0][r, pl.ds(k * 16, 16)] = jnp.zeros((16,), jnp.float32)
            return carry

        lax.fori_loop(0, CHUNK, zero_row, 0)
        for j in range(RPT // CHUNK):
            pltpu.sync_copy(rows[0],
                            acc.at[pl.ds(s * RPT + j * CHUNK, CHUNK)])
        plsc.subcore_barrier()

        def issue_idx(j, t, d):
            eb = base + j * CHUNK
            pltpu.async_copy(adj_hbm.at[pl.ds(E + eb, CHUNK)], srcs[t], sem_e)
            pltpu.async_copy(adj_hbm.at[pl.ds(eb, CHUNK)], dsts[d], sem_e)
            pltpu.async_copy(val_hbm.at[pl.ds(eb, CHUNK)], vals[t], sem_e)

        def wait_idx(t, d):
            pltpu.make_async_copy(
                adj_hbm.at[pl.ds(0, CHUNK)], srcs[t], sem_e).wait()
            pltpu.make_async_copy(
                adj_hbm.at[pl.ds(0, CHUNK)], dsts[d], sem_e).wait()
            pltpu.make_async_copy(
                val_hbm.at[pl.ds(0, CHUNK)], vals[t], sem_e).wait()

        def wait_scatter(b, d):
            for h in range(2):
                pltpu.make_async_copy(
                    rows[b].at[pl.ds(h * HALF, HALF)],
                    acc.at[dsts[d].at[pl.ds(h * HALF, HALF)]], sem_s).wait()

        def step(j, b, t, d, *, first=False, issue_next=True, prefetch=True):
            if issue_next:
                wait_idx((t + 1) % 3, (d + 1) % 3)
            if not first:
                wait_scatter(1 - b, (d + 2) % 3)
            if issue_next:
                pltpu.async_copy(
                    emb_hbm.at[srcs[(t + 1) % 3]], rows[1 - b], sem_g)
            if prefetch:
                issue_idx(j + 2, (t + 2) % 3, (d + 2) % 3)
            pltpu.make_async_copy(emb_hbm.at[srcs[t]], rows[b], sem_g).wait()

            for h in range(2):
                @plsc.parallel_loop(h * HALF, (h + 1) * HALF, 1, unroll=2)
                def scale_row(r):
                    vsplat = plsc.load_gather(
                        vals[t], [jnp.full((16,), r, jnp.int32)])
                    for k in range(D // 16):
                        sl = pl.ds(k * 16, 16)
                        rows[b][r, sl] = rows[b][r, sl] * vsplat

                pltpu.async_copy(
                    rows[b].at[pl.ds(h * HALF, HALF)],
                    acc.at[dsts[d].at[pl.ds(h * HALF, HALF)]],
                    sem_s, add=True)

        issue_idx(0, 0, 0)
        wait_idx(0, 0)
        pltpu.async_copy(emb_hbm.at[srcs[0]], rows[0], sem_g)
        issue_idx(1, 1, 1)

        step(0, 0, 0, 0, first=True)

        def six(p, carry):
            for u in range(UNROLL):
                j = (1 + u) + p * UNROLL
                step(j, (1 + u) % 2, (1 + u) % 3, (1 + u) % 3)
            return carry

        lax.fori_loop(0, BODY_ITERS, six, 0)

        for j in range(TAIL_START, NCHUNKS):
            step(j, j % 2, j % 3, j % 3,
                 issue_next=(j + 1 < NCHUNKS),
                 prefetch=(j + 2 < NCHUNKS))
        wait_scatter((NCHUNKS - 1) % 2, (NCHUNKS - 1) % 3)
        plsc.subcore_barrier()

        for j in range(RPT // CHUNK):
            start = s * RPT + j * CHUNK
            pltpu.sync_copy(acc.at[pl.ds(start, CHUNK)],
                            out_hbm.at[c, pl.ds(start, CHUNK)])

    return spmm


_COMBINE_BLK = 2000


def _combine_body(p_ref, o_ref):
    x = p_ref[0] + p_ref[1]
    o_ref[...] = jnp.where(x >= 0, x, LEAKY * x)


@functools.cache
def _build_combine():
    return pl.pallas_call(
        _combine_body,
        grid=(N // _COMBINE_BLK,),
        in_specs=[pl.BlockSpec((NC, _COMBINE_BLK, D), lambda i: (0, i, 0))],
        out_specs=pl.BlockSpec((_COMBINE_BLK, D), lambda i: (i, 0)),
        out_shape=jax.ShapeDtypeStruct((N, D), jnp.float32),
    )


def kernel(adj_indices, adj_values, embeds):
    adj_flat = adj_indices.astype(jnp.int32).reshape(-1)
    partials = _build_spmm()(adj_flat, adj_values, embeds)
    return _build_combine()(partials)


# speedup vs baseline: 1.0281x; 1.0041x over previous
import functools

import jax
import jax.numpy as jnp
from jax import lax
from jax.experimental import pallas as pl
from jax.experimental.pallas import tpu as pltpu
from jax.experimental.pallas import tpu_sc as plsc

N = 10000
E = 320000
D = 128
LEAKY = 0.2

NC = 2
NS = 16
NW = NC * NS
EPW = E // NW
CHUNK = 80
HALF = CHUNK // 2
NCHUNKS = EPW // CHUNK
NPAD = 10240
RPT = NPAD // NS
UNROLL = 6
BODY_ITERS = (NCHUNKS - 1) // UNROLL
TAIL_START = 1 + BODY_ITERS * UNROLL


@functools.cache
def _build_spmm():
    mesh = plsc.VectorSubcoreMesh(core_axis_name="c", subcore_axis_name="s")

    @functools.partial(
        pl.kernel,
        out_type=jax.ShapeDtypeStruct((NC, NPAD, D), jnp.float32),
        mesh=mesh,
        scratch_types=[
            [pltpu.VMEM((CHUNK,), jnp.int32) for _ in range(3)],
            [pltpu.VMEM((CHUNK,), jnp.int32) for _ in range(3)],
            [pltpu.VMEM((CHUNK,), jnp.float32) for _ in range(3)],
            [pltpu.VMEM((CHUNK, D), jnp.float32) for _ in range(2)],
            pltpu.VMEM_SHARED((NPAD, D), jnp.float32),
            pltpu.SemaphoreType.DMA,
            pltpu.SemaphoreType.DMA,
            pltpu.SemaphoreType.DMA,
        ],
        compiler_params=pltpu.CompilerParams(needs_layout_passes=False),
    )
    def spmm(adj_hbm, val_hbm, emb_hbm, out_hbm,
             srcs, dsts, vals, rows, acc, sem_e, sem_g, sem_s):
        c = lax.axis_index("c")
        s = lax.axis_index("s")
        base = (s * NC + c) * EPW

        def zero_row(r, carry):
            for k in range(D // 16):
                rows[0][r, pl.ds(k * 16, 16)] = jnp.zeros((16,), jnp.float32)
            return carry

        lax.fori_loop(0, CHUNK, zero_row, 0)
        for j in range(RPT // CHUNK):
            pltpu.sync_copy(rows[0],
                            acc.at[pl.ds(s * RPT + j * CHUNK, CHUNK)])
        plsc.subcore_barrier()

        def issue_idx(j, t, d):
            eb = base + j * CHUNK
            pltpu.async_copy(adj_hbm.at[pl.ds(E + eb, CHUNK)], srcs[t], sem_e)
            pltpu.async_copy(adj_hbm.at[pl.ds(eb, CHUNK)], dsts[d], sem_e)
            pltpu.async_copy(val_hbm.at[pl.ds(eb, CHUNK)], vals[t], sem_e)

        def wait_idx(t, d):
            pltpu.make_async_copy(
                adj_hbm.at[pl.ds(0, CHUNK)], srcs[t], sem_e).wait()
            pltpu.make_async_copy(
                adj_hbm.at[pl.ds(0, CHUNK)], dsts[d], sem_e).wait()
            pltpu.make_async_copy(
                val_hbm.at[pl.ds(0, CHUNK)], vals[t], sem_e).wait()

        def wait_scatter(b, d):
            for h in range(2):
                pltpu.make_async_copy(
                    rows[b].at[pl.ds(h * HALF, HALF)],
                    acc.at[dsts[d].at[pl.ds(h * HALF, HALF)]], sem_s).wait()

        def step(j, b, t, d, *, first=False, issue_next=True, prefetch=True):
            if issue_next:
                wait_idx((t + 1) % 3, (d + 1) % 3)
            if not first:
                wait_scatter(1 - b, (d + 2) % 3)
            if issue_next:
                pltpu.async_copy(
                    emb_hbm.at[srcs[(t + 1) % 3]], rows[1 - b], sem_g)
            if prefetch:
                issue_idx(j + 2, (t + 2) % 3, (d + 2) % 3)
            pltpu.make_async_copy(emb_hbm.at[srcs[t]], rows[b], sem_g).wait()

            for h in range(2):
                @plsc.parallel_loop(h * HALF, (h + 1) * HALF, 1, unroll=2)
                def scale_row(r):
                    vsplat = plsc.load_gather(
                        vals[t], [jnp.full((16,), r, jnp.int32)])
                    for k in range(D // 16):
                        sl = pl.ds(k * 16, 16)
                        rows[b][r, sl] = rows[b][r, sl] * vsplat

                pltpu.async_copy(
                    rows[b].at[pl.ds(h * HALF, HALF)],
                    acc.at[dsts[d].at[pl.ds(h * HALF, HALF)]],
                    sem_s, add=True)

        issue_idx(0, 0, 0)
        wait_idx(0, 0)
        pltpu.async_copy(emb_hbm.at[srcs[0]], rows[0], sem_g)
        issue_idx(1, 1, 1)

        step(0, 0, 0, 0, first=True)

        def six(p, carry):
            for u in range(UNROLL):
                j = (1 + u) + p * UNROLL
                step(j, (1 + u) % 2, (1 + u) % 3, (1 + u) % 3)
            return carry

        lax.fori_loop(0, BODY_ITERS, six, 0)

        for j in range(TAIL_START, NCHUNKS):
            step(j, j % 2, j % 3, j % 3,
                 issue_next=(j + 1 < NCHUNKS),
                 prefetch=(j + 2 < NCHUNKS))
        wait_scatter((NCHUNKS - 1) % 2, (NCHUNKS - 1) % 3)
        plsc.subcore_barrier()

        for j in range(RPT // CHUNK):
            start = s * RPT + j * CHUNK
            pltpu.sync_copy(acc.at[pl.ds(start, CHUNK)],
                            out_hbm.at[c, pl.ds(start, CHUNK)])

    return spmm


_COMBINE_BLK = 10000


def _combine_body(p_ref, o_ref):
    x = p_ref[0] + p_ref[1]
    o_ref[...] = jnp.where(x >= 0, x, LEAKY * x)


@functools.cache
def _build_combine():
    return pl.pallas_call(
        _combine_body,
        grid=(N // _COMBINE_BLK,),
        in_specs=[pl.BlockSpec((NC, _COMBINE_BLK, D), lambda i: (0, i, 0))],
        out_specs=pl.BlockSpec((_COMBINE_BLK, D), lambda i: (i, 0)),
        out_shape=jax.ShapeDtypeStruct((N, D), jnp.float32),
    )


def kernel(adj_indices, adj_values, embeds):
    adj_flat = adj_indices.astype(jnp.int32).reshape(-1)
    partials = _build_spmm()(adj_flat, adj_values, embeds)
    return _build_combine()(partials)
